# baseline (speedup 1.0000x reference)
import functools

import jax
import jax.numpy as jnp
from jax import lax
from jax.experimental import pallas as pl
from jax.experimental.pallas import tpu as pltpu
from jax.experimental.pallas import tpu_sc as plsc

D = 256
N_NODES = 100000
N_REG = 10000
B = 50000

NC = 2
NS = 16
NW = NC * NS

CHUNK = 112
N_CHUNKS = 14
B_W = CHUNK * N_CHUNKS
B_LAST = B - (NW - 1) * B_W
PAD = NW * B_W - B

REG_W0 = 624
REG_CHUNKS0 = (112, 112, 112, 112, 112, 64)
REG_TAIL0 = NS * REG_W0

NV = D // 16


def _kadd(state, xs):
    acc, comp = state
    new_acc, new_comp = [], []
    for a, c, x in zip(acc, comp, xs):
        y = x - c
        t = a + y
        new_comp.append((t - a) - y)
        new_acc.append(t)
    return tuple(new_acc), tuple(new_comp)


def _accum(buf_ref, nrows, state):
    def body(r, st):
        return _kadd(st, tuple(buf_ref[r, pl.ds(i * 16, 16)]
                               for i in range(NV)))
    return lax.fori_loop(0, nrows, body, state)


@functools.partial(
    pl.kernel,
    mesh=plsc.VectorSubcoreMesh(core_axis_name="c", subcore_axis_name="s"),
    out_type=jax.ShapeDtypeStruct((NW, 4 * D), jnp.float32),
    scratch_types=[
        pltpu.VMEM((B_W,), jnp.int32),
        pltpu.VMEM((CHUNK, D), jnp.float32),
        pltpu.VMEM((CHUNK, D), jnp.float32),
        pltpu.VMEM((CHUNK, D), jnp.float32),
        pltpu.VMEM((CHUNK, D), jnp.float32),
        pltpu.VMEM((1, D), jnp.float32),
        pltpu.VMEM((4 * D,), jnp.float32),
        pltpu.SemaphoreType.DMA,
        pltpu.SemaphoreType.DMA,
        pltpu.SemaphoreType.DMA,
        pltpu.SemaphoreType.DMA,
    ],
)
def _sc_sums(node_hbm, region_hbm, idx_hbm, out_hbm,
             idx_v, buf0, buf1, buf2, buf3, row_v, out_v,
             sem0, sem1, sem2, sem3):
    cidx = lax.axis_index("c")
    sidx = lax.axis_index("s")
    wid = sidx * NC + cidx
    zeros = tuple(jnp.zeros((16,), jnp.float32) for _ in range(NV))
    rbase = sidx * REG_W0

    ibase = wid * B_W

    @pl.when(wid != NW - 1)
    def _():
        pltpu.sync_copy(idx_hbm.at[pl.ds(ibase, B_W)], idx_v)

    @pl.when(wid == NW - 1)
    def _():
        pltpu.sync_copy(idx_hbm.at[pl.ds(ibase, B_LAST)],
                        idx_v.at[pl.ds(0, B_LAST)])
        zi = jnp.zeros((16,), jnp.int32)
        for t in range(PAD // 16):
            idx_v[pl.ds(B_LAST + t * 16, 16)] = zi

    bbufs = (buf0, buf1, buf2, buf3)
    bsems = (sem0, sem1, sem2, sem3)

    def gather(c, buf, sem):
        return pltpu.async_copy(
            node_hbm.at[idx_v.at[pl.ds(c * CHUNK, CHUNK)]], buf, sem)

    def gwait(buf, sem):
        pltpu.make_async_copy(
            node_hbm.at[idx_v.at[pl.ds(0, CHUNK)]], buf, sem).wait()

    for j in range(3):
        gather(j, bbufs[j], bsems[j])

    def bbody(p, st):
        c0 = p * 4
        for j in range(4):
            c = c0 + j
            jn = (j + 3) % 4
            gwait(bbufs[j], bsems[j])

            @pl.when(c + 3 < N_CHUNKS)
            def _():
                gather(c + 3, bbufs[jn], bsems[jn])

            st = _accum(bbufs[j], CHUNK, st)
        return st

    st_b = lax.fori_loop(0, (N_CHUNKS - 2) // 4, bbody, (zeros, zeros))

    zf = jnp.zeros((16,), jnp.float32)
    for i in range(NV):
        out_v[pl.ds(i * 16, 16)] = zf
        out_v[pl.ds(2 * D + i * 16, 16)] = zf

    roffs = []
    o = 0
    for n in REG_CHUNKS0:
        roffs.append(o)
        o += n
    rbufs = (buf2, buf3, buf0, buf1, buf2, buf3)
    rsems = (sem2, sem3, sem0, sem1, sem2, sem3)

    def rissue(c):
        n = REG_CHUNKS0[c]
        pltpu.async_copy(
            region_hbm.at[pl.ds(rbase + roffs[c], n)],
            rbufs[c].at[pl.ds(0, n)], rsems[c])

    def rwait(c):
        n = REG_CHUNKS0[c]
        pltpu.make_async_copy(
            region_hbm.at[pl.ds(rbase, n)],
            rbufs[c].at[pl.ds(0, n)], rsems[c]).wait()

    @pl.when(cidx == 0)
    def _():
        rissue(0)
        rissue(1)

    for c in (N_CHUNKS - 2, N_CHUNKS - 1):
        j = c % 4
        gwait(bbufs[j], bsems[j])
        st_b = _accum(bbufs[j], CHUNK, st_b)

    pltpu.sync_copy(node_hbm.at[pl.ds(0, 1)], row_v)
    w = jnp.float32(PAD / NW)
    st_b = _kadd(st_b, tuple(-w * row_v[0, pl.ds(i * 16, 16)]
                             for i in range(NV)))
    acc_b, comp_b = st_b

    @pl.when(cidx == 0)
    def _():
        st_r = (zeros, zeros)
        for c in range(len(REG_CHUNKS0)):
            if c + 2 < len(REG_CHUNKS0):
                rissue(c + 2)
            rwait(c)
            st_r = _accum(rbufs[c], REG_CHUNKS0[c], st_r)
        pltpu.sync_copy(region_hbm.at[pl.ds(REG_TAIL0 + sidx, 1)], row_v)
        st_r = _kadd(st_r, tuple(row_v[0, pl.ds(i * 16, 16)]
                                 for i in range(NV)))
        acc_r, comp_r = st_r
        for i in range(NV):
            out_v[pl.ds(i * 16, 16)] = acc_r[i]
            out_v[pl.ds(2 * D + i * 16, 16)] = -comp_r[i]

    for i in range(NV):
        out_v[pl.ds(D + i * 16, 16)] = acc_b[i]
        out_v[pl.ds(3 * D + i * 16, 16)] = -comp_b[i]
    pltpu.sync_copy(out_v, out_hbm.at[wid])


def _head_body(partials, W1, b1, W2, b2, W3, b3, W4, b4, W5, b5, W6, b6, out):
    p = partials[...]
    s = jnp.zeros((2 * D,), jnp.float32)
    c = jnp.zeros((2 * D,), jnp.float32)
    for half in (p[:, :2 * D], p[:, 2 * D:]):
        for i in range(NW):
            y = half[i] - c
            t = s + y
            c = (t - s) - y
            s = t
    gs = jnp.broadcast_to((s[:D] * jnp.float32(1.0 / N_REG)).reshape(1, D), (8, D))
    bm = jnp.broadcast_to((s[D:] * jnp.float32(1.0 / B)).reshape(1, D), (8, D))
    f32 = jnp.float32

    def dot3x(a, b):
        a_hi = a.astype(jnp.bfloat16).astype(f32)
        a_lo = a - a_hi
        b_hi = b.astype(jnp.bfloat16).astype(f32)
        b_lo = b - b_hi
        d = functools.partial(jnp.dot, preferred_element_type=f32)
        return d(a_hi, b_lo) + d(a_lo, b_hi) + d(a_hi, b_hi)

    gs = jnp.maximum(dot3x(gs, W1[...]) + b1[...], 0.0)
    gs = jnp.maximum(dot3x(gs, W2[...]) + b2[...], 0.0)
    bi = jnp.maximum(dot3x(bm, W3[...]) + b3[...], 0.0)
    bi = dot3x(bi, W4[...]) + b4[...]
    w5 = W5[...]
    h = dot3x(gs, w5[:128]) + dot3x(bi, w5[128:])
    h = jnp.maximum(h + b5[...], 0.0)
    v = dot3x(h, W6[...]) + b6[...]
    out[...] = v[0:1, 0:1]


def kernel(node_embeddings, region_embeddings, boundary_nodes,
           W1, b1, W2, b2, W3, b3, W4, b4, W5, b5, W6, b6):
    idx = boundary_nodes.astype(jnp.int32)

    partials = _sc_sums(node_embeddings, region_embeddings, idx)

    out = pl.pallas_call(
        _head_body,
        out_shape=jax.ShapeDtypeStruct((1, 1), jnp.float32),
    )(partials, W1, b1, W2, b2, W3, b3, W4, b4, W5, b5, W6, b6)
    return out.reshape(1)


# speedup vs baseline: 1.2965x; 1.2965x over previous
import functools

import jax
import jax.numpy as jnp
from jax import lax
from jax.experimental import pallas as pl
from jax.experimental.pallas import tpu as pltpu
from jax.experimental.pallas import tpu_sc as plsc

D = 256
N_NODES = 100000
N_REG = 10000
B = 50000

NC = 2
NS = 16
NW = NC * NS

CHUNK = 112
N_CHUNKS = 14
B_W = CHUNK * N_CHUNKS
B_LAST = B - (NW - 1) * B_W
PAD = NW * B_W - B

REG_W0 = 624
REG_CHUNKS0 = (112, 112, 112, 112, 112, 64)
REG_TAIL0 = NS * REG_W0

NV = D // 16


def _accum(buf_ref, nrows, acc):
    def body(r, a):
        return tuple(a[i] + buf_ref[r, pl.ds(i * 16, 16)] for i in range(NV))
    zc = tuple(jnp.zeros((16,), jnp.float32) for _ in range(NV))
    cs = lax.fori_loop(0, nrows, body, zc)
    return tuple(acc[i] + cs[i] for i in range(NV))


@functools.partial(
    pl.kernel,
    mesh=plsc.VectorSubcoreMesh(core_axis_name="c", subcore_axis_name="s"),
    out_type=jax.ShapeDtypeStruct((NW, 2 * D), jnp.float32),
    scratch_types=[
        pltpu.VMEM((B_W,), jnp.int32),
        pltpu.VMEM((CHUNK, D), jnp.float32),
        pltpu.VMEM((CHUNK, D), jnp.float32),
        pltpu.VMEM((CHUNK, D), jnp.float32),
        pltpu.VMEM((CHUNK, D), jnp.float32),
        pltpu.VMEM((1, D), jnp.float32),
        pltpu.VMEM((2 * D,), jnp.float32),
        pltpu.SemaphoreType.DMA,
        pltpu.SemaphoreType.DMA,
        pltpu.SemaphoreType.DMA,
        pltpu.SemaphoreType.DMA,
    ],
)
def _sc_sums(node_hbm, region_hbm, idx_hbm, out_hbm,
             idx_v, buf0, buf1, buf2, buf3, row_v, out_v,
             sem0, sem1, sem2, sem3):
    cidx = lax.axis_index("c")
    sidx = lax.axis_index("s")
    wid = sidx * NC + cidx
    zeros = tuple(jnp.zeros((16,), jnp.float32) for _ in range(NV))
    rbase = sidx * REG_W0

    ibase = wid * B_W

    @pl.when(wid != NW - 1)
    def _():
        pltpu.sync_copy(idx_hbm.at[pl.ds(ibase, B_W)], idx_v)

    @pl.when(wid == NW - 1)
    def _():
        pltpu.sync_copy(idx_hbm.at[pl.ds(ibase, B_LAST)],
                        idx_v.at[pl.ds(0, B_LAST)])
        zi = jnp.zeros((16,), jnp.int32)
        for t in range(PAD // 16):
            idx_v[pl.ds(B_LAST + t * 16, 16)] = zi

    bbufs = (buf0, buf1, buf2, buf3)
    bsems = (sem0, sem1, sem2, sem3)

    def gather(c, buf, sem):
        return pltpu.async_copy(
            node_hbm.at[idx_v.at[pl.ds(c * CHUNK, CHUNK)]], buf, sem)

    def gwait(buf, sem):
        pltpu.make_async_copy(
            node_hbm.at[idx_v.at[pl.ds(0, CHUNK)]], buf, sem).wait()

    for j in range(3):
        gather(j, bbufs[j], bsems[j])

    def bbody(p, a):
        c0 = p * 4
        for j in range(4):
            c = c0 + j
            jn = (j + 3) % 4
            gwait(bbufs[j], bsems[j])

            @pl.when(c + 3 < N_CHUNKS)
            def _():
                gather(c + 3, bbufs[jn], bsems[jn])

            a = _accum(bbufs[j], CHUNK, a)
        return a

    acc_b = lax.fori_loop(0, (N_CHUNKS - 2) // 4, bbody, zeros)

    zf = jnp.zeros((16,), jnp.float32)
    for i in range(NV):
        out_v[pl.ds(i * 16, 16)] = zf

    roffs = []
    o = 0
    for n in REG_CHUNKS0:
        roffs.append(o)
        o += n
    rbufs = (buf2, buf3, buf0, buf1, buf2, buf3)
    rsems = (sem2, sem3, sem0, sem1, sem2, sem3)

    def rissue(c):
        n = REG_CHUNKS0[c]
        pltpu.async_copy(
            region_hbm.at[pl.ds(rbase + roffs[c], n)],
            rbufs[c].at[pl.ds(0, n)], rsems[c])

    def rwait(c):
        n = REG_CHUNKS0[c]
        pltpu.make_async_copy(
            region_hbm.at[pl.ds(rbase, n)],
            rbufs[c].at[pl.ds(0, n)], rsems[c]).wait()

    @pl.when(cidx == 0)
    def _():
        rissue(0)
        rissue(1)

    for c in (N_CHUNKS - 2, N_CHUNKS - 1):
        j = c % 4
        gwait(bbufs[j], bsems[j])
        acc_b = _accum(bbufs[j], CHUNK, acc_b)

    pltpu.sync_copy(node_hbm.at[pl.ds(0, 1)], row_v)
    w = jnp.float32(PAD / NW)
    acc_b = tuple(acc_b[i] - w * row_v[0, pl.ds(i * 16, 16)]
                  for i in range(NV))

    @pl.when(cidx == 0)
    def _():
        acc_r = zeros
        for c in range(len(REG_CHUNKS0)):
            if c + 2 < len(REG_CHUNKS0):
                rissue(c + 2)
            rwait(c)
            acc_r = _accum(rbufs[c], REG_CHUNKS0[c], acc_r)
        pltpu.sync_copy(region_hbm.at[pl.ds(REG_TAIL0 + sidx, 1)], row_v)
        acc_r = tuple(acc_r[i] + row_v[0, pl.ds(i * 16, 16)]
                      for i in range(NV))
        for i in range(NV):
            out_v[pl.ds(i * 16, 16)] = acc_r[i]

    for i in range(NV):
        out_v[pl.ds(D + i * 16, 16)] = acc_b[i]
    pltpu.sync_copy(out_v, out_hbm.at[wid])


def _head_body(partials, W1, b1, W2, b2, W3, b3, W4, b4, W5, b5, W6, b6, out):
    p = partials[...]
    s = jnp.zeros((2 * D,), jnp.float32)
    c = jnp.zeros((2 * D,), jnp.float32)
    for i in range(NW):
        y = p[i] - c
        t = s + y
        c = (t - s) - y
        s = t
    gs = jnp.broadcast_to((s[:D] * jnp.float32(1.0 / N_REG)).reshape(1, D), (8, D))
    bm = jnp.broadcast_to((s[D:] * jnp.float32(1.0 / B)).reshape(1, D), (8, D))
    f32 = jnp.float32

    def dotd(a, b):
        return jnp.dot(a, b, preferred_element_type=f32)

    gs = jnp.maximum(dotd(gs, W1[...]) + b1[...], 0.0)
    gs = jnp.maximum(dotd(gs, W2[...]) + b2[...], 0.0)
    bi = jnp.maximum(dotd(bm, W3[...]) + b3[...], 0.0)
    bi = dotd(bi, W4[...]) + b4[...]
    w5 = W5[...]
    h = dotd(gs, w5[:128]) + dotd(bi, w5[128:])
    h = jnp.maximum(h + b5[...], 0.0)
    v = dotd(h, W6[...]) + b6[...]
    out[...] = v[0:1, 0:1]


def kernel(node_embeddings, region_embeddings, boundary_nodes,
           W1, b1, W2, b2, W3, b3, W4, b4, W5, b5, W6, b6):
    idx = boundary_nodes.astype(jnp.int32)

    partials = _sc_sums(node_embeddings, region_embeddings, idx)

    out = pl.pallas_call(
        _head_body,
        out_shape=jax.ShapeDtypeStruct((1, 1), jnp.float32),
    )(partials, W1, b1, W2, b2, W3, b3, W4, b4, W5, b5, W6, b6)
    return out.reshape(1)
